# baseline (speedup 1.0000x reference)
import functools

import jax
import jax.numpy as jnp
from jax import lax
from jax.experimental import pallas as pl
from jax.experimental.pallas import tpu as pltpu
from jax.experimental.pallas import tpu_sc as plsc

EMB = 32
N_IDX = 16384 * 50
RPB = 128
NBLK = N_IDX // RPB
NC, NS, L = 2, 16, 16
NW = NC * NS
BLK_W = NBLK // NW
K = 8
NCHUNK = BLK_W // K

_mesh = plsc.VectorSubcoreMesh(core_axis_name="c", subcore_axis_name="s")


@functools.partial(
    pl.kernel,
    out_type=jax.ShapeDtypeStruct((N_IDX, EMB), jnp.float32),
    mesh=_mesh,
    scratch_types=[
        pltpu.VMEM((K, RPB), jnp.int32),
        pltpu.VMEM((K * RPB, EMB), jnp.float32),
        pltpu.SemaphoreType.DMA,
    ],
    compiler_params=pltpu.CompilerParams(
        needs_layout_passes=False, use_tc_tiling_on_sc=False),
)
def _gather_maxnorm(idx_hbm, table_hbm, out_hbm, idx_v, rows_v, sem):
    wid = lax.axis_index("s") * NC + lax.axis_index("c")
    iota16 = lax.broadcasted_iota(jnp.int32, (L,), 0)

    def chunk_body(t, carry):
        j0 = wid * BLK_W + t * K
        pltpu.sync_copy(idx_hbm.at[pl.ds(j0, K)], idx_v)
        copies = [
            pltpu.async_copy(
                table_hbm.at[idx_v.at[j]], rows_v.at[pl.ds(j * RPB, RPB)], sem)
            for j in range(K)
        ]
        for c in copies:
            c.wait()

        def group_body(g, gcarry):
            rid = iota16 + g * L
            acc = jnp.zeros((L,), jnp.float32)
            cols = []
            for d in range(EMB):
                sd = jnp.full((L,), d, jnp.int32)
                c = plsc.load_gather(rows_v, [rid, sd])
                cols.append(c)
                acc = acc + c * c
            ibits = plsc.bitcast(acc, jnp.int32)
            ibits = jnp.int32(0x5F3759DF) - lax.shift_right_logical(ibits, 1)
            y = plsc.bitcast(ibits, jnp.float32)
            for _ in range(3):
                y = y * (jnp.float32(1.5) - jnp.float32(0.5) * acc * y * y)
            norm = acc * y
            scale = jnp.where(
                acc > 1.0, jnp.float32(1.0) / (norm + jnp.float32(1e-7)),
                jnp.float32(1.0))
            for d in range(EMB):
                sd = jnp.full((L,), d, jnp.int32)
                plsc.store_scatter(rows_v, [rid, sd], cols[d] * scale)
            return gcarry

        lax.fori_loop(0, K * RPB // L, group_body, 0)
        pltpu.sync_copy(rows_v, out_hbm.at[pl.ds(j0 * RPB, K * RPB)])
        return carry

    lax.fori_loop(0, NCHUNK, chunk_body, 0)


def kernel(indices, emb_table):
    idx = indices.reshape(NBLK, RPB).astype(jnp.int32)
    out = _gather_maxnorm(idx, emb_table)
    return out.reshape(indices.shape[0], indices.shape[1], EMB)


# speedup vs baseline: 1.3996x; 1.3996x over previous
import functools

import jax
import jax.numpy as jnp
from jax import lax
from jax.experimental import pallas as pl
from jax.experimental.pallas import tpu as pltpu
from jax.experimental.pallas import tpu_sc as plsc

NI, NJ = 16384, 50
EMB = 32
NC, NS, L = 2, 16, 16
NW = NC * NS
ROWS_W = NI // NW
G = 32
NCHUNK = ROWS_W // G
GROUPS = G * NJ // L

_mesh = plsc.VectorSubcoreMesh(core_axis_name="c", subcore_axis_name="s")


@functools.partial(
    pl.kernel,
    out_type=jax.ShapeDtypeStruct((NI, NJ, EMB), jnp.float32),
    mesh=_mesh,
    scratch_types=[
        pltpu.VMEM((G, NJ), jnp.int32),
        pltpu.VMEM((G, NJ), jnp.int32),
        pltpu.VMEM((G, NJ, EMB), jnp.float32),
        pltpu.VMEM((G, NJ, EMB), jnp.float32),
        pltpu.SemaphoreType.DMA,
        pltpu.SemaphoreType.DMA,
    ],
    compiler_params=pltpu.CompilerParams(
        needs_layout_passes=False, use_tc_tiling_on_sc=False),
)
def _gather_maxnorm(idx_hbm, table_hbm, out_hbm,
                    idx_a, idx_b, rows_a, rows_b, sem_a, sem_b):
    wid = lax.axis_index("s") * NC + lax.axis_index("c")
    row0 = wid * ROWS_W
    iota16 = lax.broadcasted_iota(jnp.int32, (L,), 0)

    def issue(t, idx_v, rows_v, sem):
        pltpu.sync_copy(idx_hbm.at[pl.ds(row0 + t * G, G)], idx_v)
        for g in range(G):
            pltpu.async_copy(table_hbm.at[idx_v.at[g]], rows_v.at[g], sem)

    def drain(idx_v, rows_v, sem):
        for g in range(G):
            pltpu.make_async_copy(
                table_hbm.at[idx_v.at[g]], rows_v.at[g], sem).wait()

    def compute_and_store(t, rows_v):
        def group_body(g, carry):
            rid = iota16 + g * L
            rdiv = rid // NJ
            rmod = rid - rdiv * NJ
            acc = jnp.zeros((L,), jnp.float32)
            cols = []
            for d in range(EMB):
                sd = jnp.full((L,), d, jnp.int32)
                c = plsc.load_gather(rows_v, [rdiv, rmod, sd])
                cols.append(c)
                acc = acc + c * c
            ibits = plsc.bitcast(acc, jnp.int32)
            ibits = jnp.int32(0x5F3759DF) - lax.shift_right_logical(ibits, 1)
            y = plsc.bitcast(ibits, jnp.float32)
            for _ in range(3):
                y = y * (jnp.float32(1.5) - jnp.float32(0.5) * acc * y * y)
            norm = acc * y
            scale = jnp.where(
                acc > 1.0, jnp.float32(1.0) / (norm + jnp.float32(1e-7)),
                jnp.float32(1.0))
            for d in range(EMB):
                sd = jnp.full((L,), d, jnp.int32)
                plsc.store_scatter(rows_v, [rdiv, rmod, sd], cols[d] * scale)
            return carry

        lax.fori_loop(0, GROUPS, group_body, 0)
        pltpu.sync_copy(rows_v, out_hbm.at[pl.ds(row0 + t * G, G)])

    issue(0, idx_a, rows_a, sem_a)

    def pair_body(p, carry):
        t0 = 2 * p
        issue(t0 + 1, idx_b, rows_b, sem_b)
        drain(idx_a, rows_a, sem_a)
        compute_and_store(t0, rows_a)

        @pl.when(p < NCHUNK // 2 - 1)
        def _():
            issue(t0 + 2, idx_a, rows_a, sem_a)

        drain(idx_b, rows_b, sem_b)
        compute_and_store(t0 + 1, rows_b)
        return carry

    lax.fori_loop(0, NCHUNK // 2, pair_body, 0)


def kernel(indices, emb_table):
    return _gather_maxnorm(indices.astype(jnp.int32), emb_table)


# speedup vs baseline: 1.7316x; 1.2372x over previous
import functools

import jax
import jax.numpy as jnp
from jax import lax
from jax.experimental import pallas as pl
from jax.experimental.pallas import tpu as pltpu
from jax.experimental.pallas import tpu_sc as plsc

NI, NJ = 16384, 50
EMB = 32
NC, NS, L = 2, 16, 16
NW = NC * NS
ROWS_W = NI // NW
G = 32
NCHUNK = ROWS_W // G
GROUPS = G * NJ // L

_mesh = plsc.VectorSubcoreMesh(core_axis_name="c", subcore_axis_name="s")


@functools.partial(
    pl.kernel,
    out_type=jax.ShapeDtypeStruct((NI, NJ, EMB), jnp.float32),
    mesh=_mesh,
    scratch_types=[
        pltpu.VMEM((G, NJ), jnp.int32),
        pltpu.VMEM((G, NJ), jnp.int32),
        pltpu.VMEM((G, NJ, EMB), jnp.float32),
        pltpu.VMEM((G, NJ, EMB), jnp.float32),
        pltpu.VMEM((L, 17), jnp.float32),
        pltpu.VMEM((L,), jnp.float32),
        pltpu.SemaphoreType.DMA,
        pltpu.SemaphoreType.DMA,
    ],
    compiler_params=pltpu.CompilerParams(
        needs_layout_passes=False, use_tc_tiling_on_sc=False),
)
def _gather_maxnorm(idx_hbm, table_hbm, out_hbm,
                    idx_a, idx_b, rows_a, rows_b, sq_v, sc_v, sem_a, sem_b):
    wid = lax.axis_index("s") * NC + lax.axis_index("c")
    row0 = wid * ROWS_W
    iota16 = lax.broadcasted_iota(jnp.int32, (L,), 0)

    def issue(t, idx_v, rows_v, sem):
        pltpu.sync_copy(idx_hbm.at[pl.ds(row0 + t * G, G)], idx_v)
        for g in range(G):
            pltpu.async_copy(table_hbm.at[idx_v.at[g]], rows_v.at[g], sem)

    def drain(idx_v, rows_v, sem):
        for g in range(G):
            pltpu.make_async_copy(
                table_hbm.at[idx_v.at[g]], rows_v.at[g], sem).wait()

    def compute_and_store(t, rows_v):
        def group_body(g, carry):
            b = g * L
            g0 = b // NJ
            j0 = b - g0 * NJ
            halves = []
            for r in range(L):
                jj = j0 + r
                wrap = (jj >= NJ).astype(jnp.int32)
                gr = g0 + wrap
                jr = jj - NJ * wrap
                a = rows_v[gr, jr, pl.ds(0, L)]
                c = rows_v[gr, jr, pl.ds(L, L)]
                halves.append((gr, jr, a, c))
                sq_v[r, pl.ds(0, L)] = a * a + c * c
            acc = jnp.zeros((L,), jnp.float32)
            for d in range(L):
                sd = jnp.full((L,), d, jnp.int32)
                acc = acc + plsc.load_gather(sq_v, [iota16, sd])
            ibits = plsc.bitcast(acc, jnp.int32)
            ibits = jnp.int32(0x5F3759DF) - lax.shift_right_logical(ibits, 1)
            y = plsc.bitcast(ibits, jnp.float32)
            for _ in range(3):
                y = y * (jnp.float32(1.5) - jnp.float32(0.5) * acc * y * y)
            norm = acc * y
            scale = jnp.where(
                acc > 1.0, jnp.float32(1.0) / (norm + jnp.float32(1e-7)),
                jnp.float32(1.0))
            sc_v[pl.ds(0, L)] = scale
            for r in range(L):
                gr, jr, a, c = halves[r]
                s = plsc.load_gather(sc_v, [jnp.full((L,), r, jnp.int32)])
                rows_v[gr, jr, pl.ds(0, L)] = a * s
                rows_v[gr, jr, pl.ds(L, L)] = c * s
            return carry

        lax.fori_loop(0, GROUPS, group_body, 0)
        pltpu.sync_copy(rows_v, out_hbm.at[pl.ds(row0 + t * G, G)])

    issue(0, idx_a, rows_a, sem_a)

    def pair_body(p, carry):
        t0 = 2 * p
        issue(t0 + 1, idx_b, rows_b, sem_b)
        drain(idx_a, rows_a, sem_a)
        compute_and_store(t0, rows_a)

        @pl.when(p < NCHUNK // 2 - 1)
        def _():
            issue(t0 + 2, idx_a, rows_a, sem_a)

        drain(idx_b, rows_b, sem_b)
        compute_and_store(t0 + 1, rows_b)
        return carry

    lax.fori_loop(0, NCHUNK // 2, pair_body, 0)


def kernel(indices, emb_table):
    return _gather_maxnorm(indices.astype(jnp.int32), emb_table)


# speedup vs baseline: 1.7836x; 1.0300x over previous
import functools

import jax
import jax.numpy as jnp
from jax import lax
from jax.experimental import pallas as pl
from jax.experimental.pallas import tpu as pltpu
from jax.experimental.pallas import tpu_sc as plsc

NI, NJ = 16384, 50
EMB = 32
NC, NS, L = 2, 16, 16
NW = NC * NS
ROWS_W = NI // NW
G = 32
NCHUNK = ROWS_W // G
GROUPS = G * NJ // L

_mesh = plsc.VectorSubcoreMesh(core_axis_name="c", subcore_axis_name="s")


@functools.partial(
    pl.kernel,
    out_type=jax.ShapeDtypeStruct((NI, NJ, EMB), jnp.float32),
    mesh=_mesh,
    scratch_types=[
        pltpu.VMEM((G, NJ), jnp.int32),
        pltpu.VMEM((G, NJ), jnp.int32),
        pltpu.VMEM((G, NJ, EMB), jnp.float32),
        pltpu.VMEM((G, NJ, EMB), jnp.float32),
        pltpu.VMEM((L, 17), jnp.float32),
        pltpu.VMEM((L,), jnp.float32),
        pltpu.SemaphoreType.DMA,
        pltpu.SemaphoreType.DMA,
    ],
    compiler_params=pltpu.CompilerParams(
        needs_layout_passes=False, use_tc_tiling_on_sc=False),
)
def _gather_maxnorm(idx_hbm, table_hbm, out_hbm,
                    idx_a, idx_b, rows_a, rows_b, sq_v, sc_v, sem_a, sem_b):
    wid = lax.axis_index("s") * NC + lax.axis_index("c")
    row0 = wid * ROWS_W
    iota16 = lax.broadcasted_iota(jnp.int32, (L,), 0)

    def issue(t, idx_v, rows_v, sem):
        pltpu.sync_copy(idx_hbm.at[pl.ds(row0 + t * G, G)], idx_v)
        for g in range(G):
            pltpu.async_copy(table_hbm.at[idx_v.at[g]], rows_v.at[g], sem)

    def drain(idx_v, rows_v, sem):
        for g in range(G):
            pltpu.make_async_copy(
                table_hbm.at[idx_v.at[g]], rows_v.at[g], sem).wait()

    def compute_and_store(t, rows_v):
        def group_body(g, carry):
            rid = iota16 + g * L
            rdiv = rid // NJ
            rmod = rid - rdiv * NJ
            acc = jnp.zeros((L,), jnp.float32)
            for d in range(EMB):
                sd = (iota16 + d) & (EMB - 1)
                c = plsc.load_gather(rows_v, [rdiv, rmod, sd])
                acc = acc + c * c
            ibits = plsc.bitcast(acc, jnp.int32)
            ibits = jnp.int32(0x5F3759DF) - lax.shift_right_logical(ibits, 1)
            y = plsc.bitcast(ibits, jnp.float32)
            for _ in range(3):
                y = y * (jnp.float32(1.5) - jnp.float32(0.5) * acc * y * y)
            norm = acc * y
            scale = jnp.where(
                acc > 1.0, jnp.float32(1.0) / (norm + jnp.float32(1e-7)),
                jnp.float32(1.0))
            for d in range(EMB):
                sd = (iota16 + d) & (EMB - 1)
                c = plsc.load_gather(rows_v, [rdiv, rmod, sd])
                plsc.store_scatter(rows_v, [rdiv, rmod, sd], c * scale)
            return carry

        lax.fori_loop(0, GROUPS, group_body, 0)
        pltpu.sync_copy(rows_v, out_hbm.at[pl.ds(row0 + t * G, G)])

    issue(0, idx_a, rows_a, sem_a)

    def pair_body(p, carry):
        t0 = 2 * p
        issue(t0 + 1, idx_b, rows_b, sem_b)
        drain(idx_a, rows_a, sem_a)
        compute_and_store(t0, rows_a)

        @pl.when(p < NCHUNK // 2 - 1)
        def _():
            issue(t0 + 2, idx_a, rows_a, sem_a)

        drain(idx_b, rows_b, sem_b)
        compute_and_store(t0 + 1, rows_b)
        return carry

    lax.fori_loop(0, NCHUNK // 2, pair_body, 0)


def kernel(indices, emb_table):
    return _gather_maxnorm(indices.astype(jnp.int32), emb_table)


# speedup vs baseline: 2.0670x; 1.1589x over previous
import functools

import jax
import jax.numpy as jnp
from jax import lax
from jax.experimental import pallas as pl
from jax.experimental.pallas import tpu as pltpu
from jax.experimental.pallas import tpu_sc as plsc

NI, NJ = 16384, 50
EMB = 32
NC, NS, L = 2, 16, 16
NW = NC * NS
ROWS_W = NI // NW
G = 32
NCHUNK = ROWS_W // G
GROUPS = G * NJ // L

_mesh = plsc.VectorSubcoreMesh(core_axis_name="c", subcore_axis_name="s")


@functools.partial(
    pl.kernel,
    out_type=jax.ShapeDtypeStruct((NI, NJ, EMB), jnp.float32),
    mesh=_mesh,
    scratch_types=[
        pltpu.VMEM((G, NJ), jnp.int32),
        pltpu.VMEM((G, NJ), jnp.int32),
        pltpu.VMEM((G, NJ, EMB), jnp.float32),
        pltpu.VMEM((G, NJ, EMB), jnp.float32),
        pltpu.SemaphoreType.DMA,
        pltpu.SemaphoreType.DMA,
        pltpu.SemaphoreType.DMA,
        pltpu.SemaphoreType.DMA,
    ],
    compiler_params=pltpu.CompilerParams(
        needs_layout_passes=False, use_tc_tiling_on_sc=False),
)
def _gather_maxnorm(idx_hbm, table_hbm, out_hbm,
                    idx_a, idx_b, rows_a, rows_b,
                    sem_a, sem_b, sem_wa, sem_wb):
    wid = lax.axis_index("s") * NC + lax.axis_index("c")
    row0 = wid * ROWS_W
    iota16 = lax.broadcasted_iota(jnp.int32, (L,), 0)

    def issue(t, idx_v, rows_v, sem):
        pltpu.sync_copy(idx_hbm.at[pl.ds(row0 + t * G, G)], idx_v)
        for g in range(G):
            pltpu.async_copy(table_hbm.at[idx_v.at[g]], rows_v.at[g], sem)

    def drain(idx_v, rows_v, sem):
        for g in range(G):
            pltpu.make_async_copy(
                table_hbm.at[idx_v.at[g]], rows_v.at[g], sem).wait()

    def compute_and_store(t, rows_v):
        def group_body(g, carry):
            rid = iota16 + g * L
            rdiv = rid // NJ
            rmod = rid - rdiv * NJ
            acc = jnp.zeros((L,), jnp.float32)
            cols = []
            for d in range(EMB):
                sd = (iota16 + d) & (EMB - 1)
                c = plsc.load_gather(rows_v, [rdiv, rmod, sd])
                cols.append(c)
                acc = acc + c * c
            ibits = plsc.bitcast(acc, jnp.int32)
            ibits = jnp.int32(0x5F3759DF) - lax.shift_right_logical(ibits, 1)
            y = plsc.bitcast(ibits, jnp.float32)
            for _ in range(3):
                y = y * (jnp.float32(1.5) - jnp.float32(0.5) * acc * y * y)
            norm = acc * y
            scale = jnp.where(
                acc > 1.0, jnp.float32(1.0) / (norm + jnp.float32(1e-7)),
                jnp.float32(1.0))
            for d in range(EMB):
                sd = (iota16 + d) & (EMB - 1)
                plsc.store_scatter(rows_v, [rdiv, rmod, sd], cols[d] * scale)
            return carry

        lax.fori_loop(0, GROUPS, group_body, 0)

    def wb_start(t, rows_v, sem_w):
        pltpu.async_copy(rows_v, out_hbm.at[pl.ds(row0 + t * G, G)], sem_w)

    def wb_wait(t, rows_v, sem_w):
        pltpu.make_async_copy(
            rows_v, out_hbm.at[pl.ds(row0 + t * G, G)], sem_w).wait()

    issue(0, idx_a, rows_a, sem_a)

    def pair_body(p, carry):
        t0 = 2 * p

        @pl.when(p > 0)
        def _():
            wb_wait(t0 - 1, rows_b, sem_wb)

        issue(t0 + 1, idx_b, rows_b, sem_b)
        drain(idx_a, rows_a, sem_a)
        compute_and_store(t0, rows_a)
        wb_start(t0, rows_a, sem_wa)
        drain(idx_b, rows_b, sem_b)
        compute_and_store(t0 + 1, rows_b)
        wb_start(t0 + 1, rows_b, sem_wb)
        wb_wait(t0, rows_a, sem_wa)

        @pl.when(p < NCHUNK // 2 - 1)
        def _():
            issue(t0 + 2, idx_a, rows_a, sem_a)

        return carry

    lax.fori_loop(0, NCHUNK // 2, pair_body, 0)
    wb_wait(NCHUNK - 1, rows_b, sem_wb)


def kernel(indices, emb_table):
    return _gather_maxnorm(indices.astype(jnp.int32), emb_table)
